# baseline (speedup 1.0000x reference)
import functools

import jax
import jax.numpy as jnp
from jax.experimental import pallas as pl
from jax.experimental.pallas import tpu as pltpu


def _body(x_ref, w1_ref, w3_ref, w2_ref, rms_ref, wf_ref, bf_ref, out_ref,
          d_ref):
    ob = pl.program_id(1)

    @pl.when(ob == 0)
    def _stage1():
        x = x_ref[...]
        a = jnp.dot(x, w1_ref[...], preferred_element_type=jnp.float32)
        c = jnp.dot(x, w3_ref[...], preferred_element_type=jnp.float32)
        h = (a * jax.nn.sigmoid(a)) * c
        dec = jnp.dot(h.astype(jnp.bfloat16), w2_ref[...],
                      preferred_element_type=jnp.float32)
        dec = dec * jax.nn.sigmoid(dec)
        dec = dec * jax.lax.rsqrt(
            jnp.mean(dec * dec, axis=-1, keepdims=True) + 1e-6)
        dec = dec * rms_ref[...]
        d_ref[...] = dec.astype(jnp.bfloat16)

    logit = jnp.dot(d_ref[...], wf_ref[...],
                    preferred_element_type=jnp.float32) + bf_ref[...]
    out_ref[...] = jax.nn.sigmoid(logit)


@functools.partial(jax.jit, static_argnames=())
def _run(x, W1, W2, W3, rms_w, Wf, bf):
    L, D = x.shape
    F = W1.shape[1]
    O = Wf.shape[1]
    BL = min(256, L)
    BO = min(1024, O)
    grid = (L // BL, O // BO)

    xb = x.astype(jnp.bfloat16)
    w1b = W1.astype(jnp.bfloat16)
    w2b = W2.astype(jnp.bfloat16)
    w3b = W3.astype(jnp.bfloat16)
    wfb = Wf.astype(jnp.bfloat16)
    rms2 = rms_w.reshape(1, D)
    bf2 = bf.reshape(1, O)

    out = pl.pallas_call(
        _body,
        grid=grid,
        in_specs=[
            pl.BlockSpec((BL, D), lambda lb, ob: (lb, 0)),
            pl.BlockSpec((D, F), lambda lb, ob: (0, 0)),
            pl.BlockSpec((D, F), lambda lb, ob: (0, 0)),
            pl.BlockSpec((F, D), lambda lb, ob: (0, 0)),
            pl.BlockSpec((1, D), lambda lb, ob: (0, 0)),
            pl.BlockSpec((D, BO), lambda lb, ob: (0, ob)),
            pl.BlockSpec((1, BO), lambda lb, ob: (0, ob)),
        ],
        out_specs=pl.BlockSpec((BL, BO), lambda lb, ob: (lb, ob)),
        out_shape=jax.ShapeDtypeStruct((L, O), jnp.float32),
        scratch_shapes=[pltpu.VMEM((BL, D), jnp.bfloat16)],
        compiler_params=pltpu.CompilerParams(
            dimension_semantics=("arbitrary", "arbitrary"),
        ),
    )(xb, w1b, w3b, w2b, rms2, wfb, bf2)
    return out


def kernel(inputs, label, W1, W2, W3, rms_w, Wf, bf):
    del label
    x = inputs[0]
    out = _run(x, W1, W2, W3, rms_w, Wf, bf)
    return out[None]


# speedup vs baseline: 1.1137x; 1.1137x over previous
import functools

import jax
import jax.numpy as jnp
from jax.experimental import pallas as pl
from jax.experimental.pallas import tpu as pltpu


def _body(x_ref, w13_ref, w2_ref, rms_ref, wf_ref, bf_ref, out_ref,
          d_ref):
    ob = pl.program_id(1)

    @pl.when(ob == 0)
    def _stage1():
        x = x_ref[...]
        F = w2_ref.shape[0]
        y = jnp.dot(x, w13_ref[...], preferred_element_type=jnp.float32)
        a = y[:, :F]
        c = y[:, F:]
        h = (a * jax.nn.sigmoid(a)) * c
        dec = jnp.dot(h.astype(jnp.bfloat16), w2_ref[...],
                      preferred_element_type=jnp.float32)
        dec = dec * jax.nn.sigmoid(dec)
        dec = dec * jax.lax.rsqrt(
            jnp.mean(dec * dec, axis=-1, keepdims=True) + 1e-6)
        dec = dec * rms_ref[...]
        d_ref[...] = dec.astype(jnp.bfloat16)

    logit = jnp.dot(d_ref[...], wf_ref[...],
                    preferred_element_type=jnp.float32) + bf_ref[...]
    out_ref[...] = jax.nn.sigmoid(logit)


@functools.partial(jax.jit, static_argnames=())
def _run(x, W1, W2, W3, rms_w, Wf, bf):
    L, D = x.shape
    F = W1.shape[1]
    O = Wf.shape[1]
    BL = min(256, L)
    BO = min(2048, O)
    grid = (L // BL, O // BO)

    xb = x.astype(jnp.bfloat16)
    w13b = jnp.concatenate(
        [W1.astype(jnp.bfloat16), W3.astype(jnp.bfloat16)], axis=1)
    w2b = W2.astype(jnp.bfloat16)
    wfb = Wf.astype(jnp.bfloat16)
    rms2 = rms_w.reshape(1, D)
    bf2 = bf.reshape(1, O)

    out = pl.pallas_call(
        _body,
        grid=grid,
        in_specs=[
            pl.BlockSpec((BL, D), lambda lb, ob: (lb, 0)),
            pl.BlockSpec((D, 2 * F), lambda lb, ob: (0, 0)),
            pl.BlockSpec((F, D), lambda lb, ob: (0, 0)),
            pl.BlockSpec((1, D), lambda lb, ob: (0, 0)),
            pl.BlockSpec((D, BO), lambda lb, ob: (0, ob)),
            pl.BlockSpec((1, BO), lambda lb, ob: (0, ob)),
        ],
        out_specs=pl.BlockSpec((BL, BO), lambda lb, ob: (lb, ob)),
        out_shape=jax.ShapeDtypeStruct((L, O), jnp.float32),
        scratch_shapes=[pltpu.VMEM((BL, D), jnp.bfloat16)],
        compiler_params=pltpu.CompilerParams(
            dimension_semantics=("arbitrary", "arbitrary"),
        ),
    )(xb, w13b, w2b, rms2, wfb, bf2)
    return out


def kernel(inputs, label, W1, W2, W3, rms_w, Wf, bf):
    del label
    x = inputs[0]
    out = _run(x, W1, W2, W3, rms_w, Wf, bf)
    return out[None]


# speedup vs baseline: 1.1752x; 1.0553x over previous
import functools

import jax
import jax.numpy as jnp
from jax.experimental import pallas as pl
from jax.experimental.pallas import tpu as pltpu


def _sigmoid(x):
    return 0.5 * jnp.tanh(0.5 * x) + 0.5


def _body(x_ref, w13_ref, w2_ref, rms_ref, wf_ref, bf_ref, out_ref, d_ref):
    ob = pl.program_id(1)
    BO = out_ref.shape[1]
    F = w2_ref.shape[0]

    @pl.when(ob == 0)
    def _stage1():
        x = x_ref[...]
        a = jnp.dot(x, w13_ref[:, :F], preferred_element_type=jnp.float32)
        c = jnp.dot(x, w13_ref[:, F:], preferred_element_type=jnp.float32)
        h = (a * _sigmoid(a)) * c
        dec = jnp.dot(h.astype(jnp.bfloat16), w2_ref[...],
                      preferred_element_type=jnp.float32)
        dec = dec * _sigmoid(dec)
        dec = dec * jax.lax.rsqrt(
            jnp.mean(dec * dec, axis=-1, keepdims=True) + 1e-6)
        dec = dec * rms_ref[...]
        d_ref[...] = dec.astype(jnp.bfloat16)

    logit = jnp.dot(d_ref[...], wf_ref[:, pl.ds(ob * BO, BO)],
                    preferred_element_type=jnp.float32)
    logit = logit + bf_ref[:, pl.ds(ob * BO, BO)]
    out_ref[...] = _sigmoid(logit)


@jax.jit
def _run(x, W1, W2, W3, rms_w, Wf, bf):
    L, D = x.shape
    F = W1.shape[1]
    O = Wf.shape[1]
    BL = min(256, L)
    BO = min(1024, O)
    grid = (L // BL, O // BO)

    xb = x.astype(jnp.bfloat16)
    w13b = jnp.concatenate(
        [W1.astype(jnp.bfloat16), W3.astype(jnp.bfloat16)], axis=1)
    w2b = W2.astype(jnp.bfloat16)
    wfb = Wf.astype(jnp.bfloat16)
    rms2 = rms_w.reshape(1, D)
    bf2 = bf.reshape(1, O)

    out = pl.pallas_call(
        _body,
        grid=grid,
        in_specs=[
            pl.BlockSpec((BL, D), lambda lb, ob: (lb, 0)),
            pl.BlockSpec((D, 2 * F), lambda lb, ob: (0, 0)),
            pl.BlockSpec((F, D), lambda lb, ob: (0, 0)),
            pl.BlockSpec((1, D), lambda lb, ob: (0, 0)),
            pl.BlockSpec((D, O), lambda lb, ob: (0, 0)),
            pl.BlockSpec((1, O), lambda lb, ob: (0, 0)),
        ],
        out_specs=pl.BlockSpec((BL, BO), lambda lb, ob: (lb, ob)),
        out_shape=jax.ShapeDtypeStruct((L, O), jnp.float32),
        scratch_shapes=[pltpu.VMEM((BL, D), jnp.bfloat16)],
        compiler_params=pltpu.CompilerParams(
            dimension_semantics=("arbitrary", "arbitrary"),
        ),
    )(xb, w13b, w2b, rms2, wfb, bf2)
    return out


def kernel(inputs, label, W1, W2, W3, rms_w, Wf, bf):
    del label
    x = inputs[0]
    out = _run(x, W1, W2, W3, rms_w, Wf, bf)
    return out[None]


# speedup vs baseline: 1.5296x; 1.3015x over previous
import jax
import jax.numpy as jnp
from jax.experimental import pallas as pl
from jax.experimental.pallas import tpu as pltpu


def _sigmoid(x):
    return 0.5 * jnp.tanh(0.5 * x) + 0.5


def _ffn_body(x_ref, w13_ref, w2_ref, rms_ref, wf_ref, d_ref, wfb_ref):
    F = w2_ref.shape[0]
    x = x_ref[...].astype(jnp.bfloat16)
    a = jnp.dot(x, w13_ref[:, :F], preferred_element_type=jnp.float32)
    c = jnp.dot(x, w13_ref[:, F:], preferred_element_type=jnp.float32)
    h = (a * _sigmoid(a)) * c
    dec = jnp.dot(h.astype(jnp.bfloat16), w2_ref[...],
                  preferred_element_type=jnp.float32)
    dec = dec * _sigmoid(dec)
    dec = dec * jax.lax.rsqrt(
        jnp.mean(dec * dec, axis=-1, keepdims=True) + 1e-6)
    dec = dec * rms_ref[...]
    d_ref[...] = dec.astype(jnp.bfloat16)
    wfb_ref[...] = wf_ref[...].astype(jnp.bfloat16)


def _out_body(d_ref, wfb_ref, bias_ref, out_ref):
    ob = pl.program_id(1)
    BO = out_ref.shape[1]
    logit = jnp.dot(d_ref[...], wfb_ref[:, pl.ds(ob * BO, BO)],
                    preferred_element_type=jnp.float32)
    logit = logit + bias_ref[:, pl.ds(ob * BO, BO)]
    out_ref[...] = _sigmoid(logit)


@jax.jit
def _run(x, W1, W2, W3, rms_w, Wf, bf):
    L, D = x.shape
    F = W1.shape[1]
    O = Wf.shape[1]

    w13b = jnp.concatenate(
        [W1.astype(jnp.bfloat16), W3.astype(jnp.bfloat16)], axis=1)
    w2b = W2.astype(jnp.bfloat16)
    rms2 = rms_w.reshape(1, D)
    bf2 = bf.reshape(1, O)

    BL_A = min(256, L)
    grid_a = L // BL_A
    WFC = O // grid_a

    d, wfb = pl.pallas_call(
        _ffn_body,
        grid=(grid_a,),
        in_specs=[
            pl.BlockSpec((BL_A, D), lambda i: (i, 0)),
            pl.BlockSpec((D, 2 * F), lambda i: (0, 0)),
            pl.BlockSpec((F, D), lambda i: (0, 0)),
            pl.BlockSpec((1, D), lambda i: (0, 0)),
            pl.BlockSpec((D, WFC), lambda i: (0, i)),
        ],
        out_specs=[
            pl.BlockSpec((BL_A, D), lambda i: (i, 0)),
            pl.BlockSpec((D, WFC), lambda i: (0, i)),
        ],
        out_shape=[
            jax.ShapeDtypeStruct((L, D), jnp.bfloat16),
            jax.ShapeDtypeStruct((D, O), jnp.bfloat16),
        ],
        compiler_params=pltpu.CompilerParams(
            dimension_semantics=("arbitrary",),
        ),
    )(x, w13b, w2b, rms2, Wf)

    BL_B = min(1024, L)
    BO_B = min(2048, O)
    out = pl.pallas_call(
        _out_body,
        grid=(L // BL_B, O // BO_B),
        in_specs=[
            pl.BlockSpec((BL_B, D), lambda lb, ob: (lb, 0)),
            pl.BlockSpec((D, O), lambda lb, ob: (0, 0)),
            pl.BlockSpec((1, O), lambda lb, ob: (0, 0)),
        ],
        out_specs=pl.BlockSpec((BL_B, BO_B), lambda lb, ob: (lb, ob)),
        out_shape=jax.ShapeDtypeStruct((L, O), jnp.float32),
        compiler_params=pltpu.CompilerParams(
            dimension_semantics=("arbitrary", "arbitrary"),
        ),
    )(d, wfb, bf2)
    return out


def kernel(inputs, label, W1, W2, W3, rms_w, Wf, bf):
    del label
    x = inputs[0]
    out = _run(x, W1, W2, W3, rms_w, Wf, bf)
    return out[None]
